# baseline (speedup 1.0000x reference)
import functools

import jax
import jax.numpy as jnp
from jax.experimental import pallas as pl
from jax.experimental.pallas import tpu as pltpu
from jax.experimental.pallas import tpu_sc as plsc

B, T, D, K = 16, 1024, 64, 1024
N = B * T
R = 1024
NT = N // R

_SC_INFO = plsc.get_sparse_core_info()
_NC = _SC_INFO.num_cores
_NW = _NC * _SC_INFO.num_subcores
_BPW = N // _NW
_CHUNK = 128
_NCHUNK = _BPW // _CHUNK


def _vq_tile(x_ref, cbt2_ref, idx_ref, loss_ref):
    i = pl.program_id(0)

    @pl.when(i == 0)
    def _init():
        loss_ref[0, 0] = jnp.float32(0.0)

    x = x_ref[...]
    xn = jnp.sum(x * x, axis=1, keepdims=True)
    cbt2 = cbt2_ref[...]
    cn = 0.25 * jnp.sum(cbt2 * cbt2, axis=0, keepdims=True)
    dot2 = jnp.dot(x, cbt2, preferred_element_type=jnp.float32)
    dist = (xn + cn) - dot2
    minv = jnp.min(dist, axis=1, keepdims=True)
    iota = jax.lax.broadcasted_iota(jnp.int32, (1, K), 1).astype(jnp.float32)
    candf = jnp.where(dist == minv, iota, jnp.float32(K))
    idxf = jnp.min(candf, axis=1)
    idx_ref[0, 0, :] = idxf.astype(jnp.int32)
    loss_ref[0, 0] += jnp.sum(minv)


def _argmin_one(xf, cbt2):
    return pl.pallas_call(
        _vq_tile,
        grid=(NT,),
        in_specs=[
            pl.BlockSpec((R, D), lambda i: (i, 0)),
            pl.BlockSpec((D, K), lambda i: (0, 0)),
        ],
        out_specs=[
            pl.BlockSpec((1, 1, R), lambda i: (i, 0, 0)),
            pl.BlockSpec(memory_space=pltpu.SMEM, block_shape=(1, 1),
                         index_map=lambda i: (0, 0)),
        ],
        out_shape=[
            jax.ShapeDtypeStruct((NT, 1, R), jnp.int32),
            jax.ShapeDtypeStruct((1, 1), jnp.float32),
        ],
    )(xf, cbt2)


def _gather_body(cb_hbm, idx_hbm, out_hbm, idx_v, rows_v, sem):
    wid = jax.lax.axis_index("s") * _NC + jax.lax.axis_index("c")
    pltpu.sync_copy(idx_hbm.at[pl.ds(wid * _NCHUNK, _NCHUNK)], idx_v)
    for j in range(_NCHUNK):
        pltpu.async_copy(cb_hbm.at[idx_v.at[j]],
                         rows_v.at[pl.ds(j * _CHUNK, _CHUNK)], sem)
    for j in range(_NCHUNK):
        pltpu.make_async_copy(cb_hbm.at[idx_v.at[j]],
                              rows_v.at[pl.ds(j * _CHUNK, _CHUNK)], sem).wait()
    pltpu.sync_copy(rows_v, out_hbm.at[pl.ds(wid * _BPW, _BPW)])


_DP = 128

_sc_gather = functools.partial(
    pl.kernel,
    mesh=plsc.VectorSubcoreMesh(core_axis_name="c", subcore_axis_name="s"),
    out_type=jax.ShapeDtypeStruct((N, _DP), jnp.float32),
    scratch_types=[
        pltpu.VMEM((_NCHUNK, _CHUNK), jnp.int32),
        pltpu.VMEM((_BPW, _DP), jnp.float32),
        pltpu.SemaphoreType.DMA,
    ],
)(_gather_body)


def kernel(feat_H, feat_L, feat_O, feat_M, feat_N, cb_H, cb_L, cb_O, cb_M, cb_N):
    feats = [feat_H, feat_L, feat_O, feat_M, feat_N]
    cbs = [cb_H, cb_L, cb_O, cb_M, cb_N]
    quantized, indices = [], []
    total_loss = jnp.float32(0.0)
    for f, cb in zip(feats, cbs):
        xf = f.reshape(N, D)
        cbt2 = 2.0 * cb.T
        idx3, loss_sum = _argmin_one(xf, cbt2)
        idx2 = idx3.reshape(N // _CHUNK, _CHUNK)
        cbp = jnp.pad(cb, ((0, 0), (0, _DP - D)))
        q = _sc_gather(cbp, idx2)
        quantized.append(q[:, :D].reshape(B, T, D))
        indices.append(idx3.reshape(B, T))
        total_loss = total_loss + loss_sum[0, 0] / jnp.float32(N * D)
    return (tuple(quantized), tuple(indices), total_loss)


# speedup vs baseline: 1.1454x; 1.1454x over previous
import functools

import jax
import jax.numpy as jnp
from jax.experimental import pallas as pl
from jax.experimental.pallas import tpu as pltpu
from jax.experimental.pallas import tpu_sc as plsc

B, T, D, K = 16, 1024, 64, 1024
N = B * T
R = 1024
NT = N // R

_SC_INFO = plsc.get_sparse_core_info()
_NC = _SC_INFO.num_cores
_NW = _NC * _SC_INFO.num_subcores
_BPW = N // _NW
_CHUNK = 128
_NCHUNK = _BPW // _CHUNK


def _vq_tile(x_ref, cbt2_ref, idx_ref, loss_ref):
    i = pl.program_id(0)

    @pl.when(i == 0)
    def _init():
        loss_ref[0, 0] = jnp.float32(0.0)

    x = x_ref[...]
    xn = jnp.sum(x * x, axis=1, keepdims=True)
    cbt2 = cbt2_ref[...]
    cn = 0.25 * jnp.sum(cbt2 * cbt2, axis=0, keepdims=True)
    dot2 = jnp.dot(x, cbt2, preferred_element_type=jnp.float32)
    dist = (xn + cn) - dot2
    minv = jnp.min(dist, axis=1, keepdims=True)
    iota = jax.lax.broadcasted_iota(jnp.int32, (1, K), 1).astype(jnp.float32)
    candf = jnp.where(dist == minv, iota, jnp.float32(K))
    idxf = jnp.min(candf, axis=1, keepdims=True)
    idx_ref[...] = idxf.astype(jnp.int32)
    loss_ref[0, 0] += jnp.sum(minv)


def _argmin_one(xf, cbt2):
    return pl.pallas_call(
        _vq_tile,
        grid=(NT,),
        in_specs=[
            pl.BlockSpec((R, D), lambda i: (i, 0)),
            pl.BlockSpec((D, K), lambda i: (0, 0)),
        ],
        out_specs=[
            pl.BlockSpec((R, 1), lambda i: (i, 0)),
            pl.BlockSpec(memory_space=pltpu.SMEM, block_shape=(1, 1),
                         index_map=lambda i: (0, 0)),
        ],
        out_shape=[
            jax.ShapeDtypeStruct((N, 1), jnp.int32),
            jax.ShapeDtypeStruct((1, 1), jnp.float32),
        ],
    )(xf, cbt2)


def _gather_body(cb_hbm, idx_hbm, out_hbm, idx_v, rows_v, sem):
    wid = jax.lax.axis_index("s") * _NC + jax.lax.axis_index("c")
    pltpu.sync_copy(idx_hbm.at[pl.ds(wid * _NCHUNK, _NCHUNK)], idx_v)
    for j in range(_NCHUNK):
        pltpu.async_copy(cb_hbm.at[idx_v.at[j]],
                         rows_v.at[pl.ds(j * _CHUNK, _CHUNK)], sem)
    for j in range(_NCHUNK):
        pltpu.make_async_copy(cb_hbm.at[idx_v.at[j]],
                              rows_v.at[pl.ds(j * _CHUNK, _CHUNK)], sem).wait()
    pltpu.sync_copy(rows_v, out_hbm.at[pl.ds(wid * _BPW, _BPW)])


_DP = 128

_sc_gather = functools.partial(
    pl.kernel,
    mesh=plsc.VectorSubcoreMesh(core_axis_name="c", subcore_axis_name="s"),
    out_type=jax.ShapeDtypeStruct((N, _DP), jnp.float32),
    scratch_types=[
        pltpu.VMEM((_NCHUNK, _CHUNK), jnp.int32),
        pltpu.VMEM((_BPW, _DP), jnp.float32),
        pltpu.SemaphoreType.DMA,
    ],
)(_gather_body)


def kernel(feat_H, feat_L, feat_O, feat_M, feat_N, cb_H, cb_L, cb_O, cb_M, cb_N):
    feats = [feat_H, feat_L, feat_O, feat_M, feat_N]
    cbs = [cb_H, cb_L, cb_O, cb_M, cb_N]
    quantized, indices = [], []
    total_loss = jnp.float32(0.0)
    for f, cb in zip(feats, cbs):
        xf = f.reshape(N, D)
        cbt2 = 2.0 * cb.T
        idx3, loss_sum = _argmin_one(xf, cbt2)
        idx2 = idx3.reshape(N // _CHUNK, _CHUNK)
        cbp = jnp.pad(cb, ((0, 0), (0, _DP - D)))
        q = _sc_gather(cbp, idx2)
        quantized.append(q[:, :D].reshape(B, T, D))
        indices.append(idx3.reshape(B, T))
        total_loss = total_loss + loss_sum[0, 0] / jnp.float32(N * D)
    return (tuple(quantized), tuple(indices), total_loss)


# speedup vs baseline: 1.1999x; 1.0476x over previous
import functools

import jax
import jax.numpy as jnp
from jax.experimental import pallas as pl
from jax.experimental.pallas import tpu as pltpu
from jax.experimental.pallas import tpu_sc as plsc

B, T, D, K = 16, 1024, 64, 1024
N = B * T
R = 2048
NT = N // R

_SC_INFO = plsc.get_sparse_core_info()
_NC = _SC_INFO.num_cores
_NW = _NC * _SC_INFO.num_subcores
_BPW = N // _NW
_CHUNK = 128
_NCHUNK = _BPW // _CHUNK


def _vq_tile(x_ref, cbt2_ref, idx_ref, loss_ref):
    i = pl.program_id(0)

    @pl.when(i == 0)
    def _init():
        loss_ref[0, 0] = jnp.float32(0.0)

    x = x_ref[...]
    xn = jnp.sum(x * x, axis=1, keepdims=True)
    cbt2 = cbt2_ref[...]
    cn = 0.25 * jnp.sum(cbt2 * cbt2, axis=0, keepdims=True)
    dot2 = jnp.dot(x, cbt2, preferred_element_type=jnp.float32)
    dist = (xn + cn) - dot2
    minv = jnp.min(dist, axis=1, keepdims=True)
    iota = jax.lax.broadcasted_iota(jnp.int32, (1, K), 1).astype(jnp.float32)
    candf = jnp.where(dist == minv, iota, jnp.float32(K))
    idxf = jnp.min(candf, axis=1, keepdims=True)
    idx_ref[...] = idxf.astype(jnp.int32)
    loss_ref[0, 0] += jnp.sum(minv)


def _argmin_one(xf, cbt2):
    return pl.pallas_call(
        _vq_tile,
        grid=(NT,),
        in_specs=[
            pl.BlockSpec((R, D), lambda i: (i, 0)),
            pl.BlockSpec((D, K), lambda i: (0, 0)),
        ],
        out_specs=[
            pl.BlockSpec((R, 1), lambda i: (i, 0)),
            pl.BlockSpec(memory_space=pltpu.SMEM, block_shape=(1, 1),
                         index_map=lambda i: (0, 0)),
        ],
        out_shape=[
            jax.ShapeDtypeStruct((N, 1), jnp.int32),
            jax.ShapeDtypeStruct((1, 1), jnp.float32),
        ],
    )(xf, cbt2)


def _gather_body(cb_hbm, idx_hbm, out_hbm, idx_v, rows_v, sem):
    wid = jax.lax.axis_index("s") * _NC + jax.lax.axis_index("c")
    pltpu.sync_copy(idx_hbm.at[pl.ds(wid * _NCHUNK, _NCHUNK)], idx_v)
    for j in range(_NCHUNK):
        pltpu.async_copy(cb_hbm.at[idx_v.at[j]],
                         rows_v.at[pl.ds(j * _CHUNK, _CHUNK)], sem)
    for j in range(_NCHUNK):
        pltpu.make_async_copy(cb_hbm.at[idx_v.at[j]],
                              rows_v.at[pl.ds(j * _CHUNK, _CHUNK)], sem).wait()
    pltpu.sync_copy(rows_v, out_hbm.at[pl.ds(wid * _BPW, _BPW)])


_DP = 128

_sc_gather = functools.partial(
    pl.kernel,
    mesh=plsc.VectorSubcoreMesh(core_axis_name="c", subcore_axis_name="s"),
    out_type=jax.ShapeDtypeStruct((N, _DP), jnp.float32),
    scratch_types=[
        pltpu.VMEM((_NCHUNK, _CHUNK), jnp.int32),
        pltpu.VMEM((_BPW, _DP), jnp.float32),
        pltpu.SemaphoreType.DMA,
    ],
)(_gather_body)


def kernel(feat_H, feat_L, feat_O, feat_M, feat_N, cb_H, cb_L, cb_O, cb_M, cb_N):
    feats = [feat_H, feat_L, feat_O, feat_M, feat_N]
    cbs = [cb_H, cb_L, cb_O, cb_M, cb_N]
    quantized, indices = [], []
    total_loss = jnp.float32(0.0)
    for f, cb in zip(feats, cbs):
        xf = f.reshape(N, D)
        cbt2 = 2.0 * cb.T
        idx3, loss_sum = _argmin_one(xf, cbt2)
        idx2 = idx3.reshape(N // _CHUNK, _CHUNK)
        cbp = jnp.pad(cb, ((0, 0), (0, _DP - D)))
        q = _sc_gather(cbp, idx2)
        quantized.append(q[:, :D].reshape(B, T, D))
        indices.append(idx3.reshape(B, T))
        total_loss = total_loss + loss_sum[0, 0] / jnp.float32(N * D)
    return (tuple(quantized), tuple(indices), total_loss)
